# baseline (speedup 1.0000x reference)
import functools

import jax
import jax.numpy as jnp
from jax import lax
from jax.experimental import pallas as pl
from jax.experimental.pallas import tpu as pltpu
from jax.experimental.pallas import tpu_sc as plsc

_NC = 2
_NS = 16
_NW = _NC * _NS
_RPD = 128


def _tc_build_p(tableT, wt, b2, vp, vb):
    hid, _ = tableT.shape
    emb = wt.shape[1]
    grid = vp // vb

    def pk(t_ref, w_ref, b_ref, o_ref):
        o_ref[...] = (
            lax.dot_general(t_ref[...], w_ref[...],
                            (((0,), (0,)), ((), ())),
                            preferred_element_type=jnp.float32)
            + b_ref[...]
        )

    return pl.pallas_call(
        pk,
        grid=(grid,),
        in_specs=[
            pl.BlockSpec((hid, vb), lambda i: (0, i)),
            pl.BlockSpec((hid, emb), lambda i: (0, 0)),
            pl.BlockSpec((1, emb), lambda i: (0, 0)),
        ],
        out_specs=pl.BlockSpec((vb, emb), lambda i: (i, 0)),
        out_shape=jax.ShapeDtypeStruct((vp, emb), jnp.float32),
    )(tableT, wt, b2)


def _sc_gather(x3, p):
    n_rows = x3.shape[0]
    emb = p.shape[1]
    rows_per_w = n_rows // _NW
    R = 2
    steps = rows_per_w // R
    half = steps // 2

    mesh = plsc.VectorSubcoreMesh(
        core_axis_name="c", subcore_axis_name="s",
        num_cores=_NC, num_subcores=_NS)

    @functools.partial(
        pl.kernel,
        out_type=jax.ShapeDtypeStruct((n_rows, _RPD, emb), jnp.float32),
        mesh=mesh,
        scratch_types=[
            pltpu.VMEM((rows_per_w, _RPD), jnp.int32),
            pltpu.VMEM((2, R, _RPD, emb), jnp.float32),
            pltpu.SemaphoreType.DMA,
            pltpu.SemaphoreType.DMA,
            pltpu.SemaphoreType.DMA,
            pltpu.SemaphoreType.DMA,
        ],
    )
    def gather_k(x_hbm, p_hbm, out_hbm, idx_v, rows_v, sg0, sg1, sw0, sw1):
        wid = lax.axis_index("s") * _NC + lax.axis_index("c")
        base = wid * rows_per_w
        sg = (sg0, sg1)
        sw = (sw0, sw1)

        pltpu.sync_copy(x_hbm.at[pl.ds(base, rows_per_w)], idx_v)

        def fire(step, b):
            for j in range(R):
                pltpu.make_async_copy(
                    p_hbm.at[idx_v.at[step * R + j]], rows_v.at[b, j], sg[b]
                ).start()

        def drain_gather(step, b):
            for j in range(R):
                pltpu.make_async_copy(
                    p_hbm.at[idx_v.at[step * R + j]], rows_v.at[b, j], sg[b]
                ).wait()

        def wb_start(step, b):
            row0 = base + step * R
            pltpu.make_async_copy(
                rows_v.at[b], out_hbm.at[pl.ds(row0, R)], sw[b]
            ).start()

        def wb_wait(step, b):
            row0 = base + step * R
            pltpu.make_async_copy(
                rows_v.at[b], out_hbm.at[pl.ds(row0, R)], sw[b]
            ).wait()

        fire(0, 0)
        fire(1, 1)

        def body(i, carry):
            g = 2 * i
            drain_gather(g, 0)
            wb_start(g, 0)
            drain_gather(g + 1, 1)
            wb_start(g + 1, 1)
            wb_wait(g, 0)
            @pl.when(i < half - 1)
            def _():
                fire(g + 2, 0)
            wb_wait(g + 1, 1)
            @pl.when(i < half - 1)
            def _():
                fire(g + 3, 1)
            return carry

        lax.fori_loop(0, half, body, 0)

    return gather_k(x3, p)


def kernel(x, table, W, b):
    bsz, seq = x.shape
    n = bsz * seq
    emb, hid = W.shape
    vocab = table.shape[0]
    vb = 32768
    vp = ((vocab + vb - 1) // vb) * vb
    p = _tc_build_p(table.T, W.T, b.reshape(1, emb), vp, vb)
    x3 = x.astype(jnp.int32).reshape(n // _RPD, _RPD)
    out = _sc_gather(x3, p)
    return out.reshape(bsz, seq, emb)
